# baseline (speedup 1.0000x reference)
import jax
import jax.numpy as jnp
from jax import lax
from jax.experimental import pallas as pl
from jax.experimental.pallas import tpu as pltpu
from jax.experimental.pallas import tpu_sc as plsc

_T = 32768
_E = 64
_L = 16
_NC = 2
_NS = 16
_NW = _NC * _NS
_TPW = _T // _NW
_CH = 256
_NCH = _TPW // _CH
_G = _CH // _L
_EB = 8
_NGA = 4


def _router_sc_body(xt_hbm, combt_hbm, idx_hbm, part_hbm,
                    in_v, ex_v, out_v, idx_v, si_v, part_v):
    wid = lax.axis_index("s") * _NC + lax.axis_index("c")
    base = wid * _TPW

    lane = lax.iota(jnp.int32, _L)
    big = jnp.full((_L,), jnp.int32(_E), jnp.int32)
    onev = jnp.full((_L,), jnp.float32(1.0), jnp.float32)
    zerov = jnp.zeros((_L,), jnp.float32)
    ecs = [jnp.full((_L,), jnp.int32(e), jnp.int32) for e in range(_E)]

    for r in range(_L):
        for k in range(2 * _E // _L):
            part_v[r, pl.ds(k * _L, _L)] = zerov

    def chunk(c, _):
        tok0 = base + c * _CH
        pltpu.sync_copy(xt_hbm.at[:, pl.ds(tok0, _CH)], in_v)

        def phase_a(gq, _):
            for q in range(_NGA):
                g = gq * _NGA + q
                t0 = g * _L
                part = []
                for h in range(4):
                    mh = in_v[16 * h, pl.ds(t0, _L)]
                    for e in range(16 * h + 1, 16 * h + 16):
                        mh = jnp.maximum(mh, in_v[e, pl.ds(t0, _L)])
                    part.append(mh)
                m = jnp.maximum(jnp.maximum(part[0], part[1]),
                                jnp.maximum(part[2], part[3]))
                ss, bb = [], []
                for h in range(4):
                    sh = zerov
                    bh = big
                    for e in range(16 * h, 16 * h + 16):
                        v = in_v[e, pl.ds(t0, _L)]
                        ex = jnp.exp(v - m)
                        ex_v[e, pl.ds(t0, _L)] = ex
                        sh = sh + ex
                        bh = jnp.minimum(bh, jnp.where(v == m, ecs[e], big))
                    ss.append(sh)
                    bb.append(bh)
                s = (ss[0] + ss[1]) + (ss[2] + ss[3])
                bi = jnp.minimum(jnp.minimum(bb[0], bb[1]),
                                 jnp.minimum(bb[2], bb[3]))
                si_v[g, :] = onev / s
                idx_v[pl.ds(t0, _L)] = bi
            return 0

        lax.fori_loop(0, _G // _NGA, phase_a, 0)

        for eb in range(_E // _EB):
            def phase_b(g, acc):
                t0 = g * _L
                bi = idx_v[pl.ds(t0, _L)]
                sinv = si_v[g, :]
                acc2 = []
                for k in range(_EB):
                    e = eb * _EB + k
                    pk, ck = acc[k]
                    ex = ex_v[e, pl.ds(t0, _L)]
                    hit = bi == ecs[e]
                    out_v[e, pl.ds(t0, _L)] = jnp.where(hit, sinv, zerov)
                    acc2.append((pk + ex * sinv,
                                 ck + jnp.where(hit, onev, zerov)))
                return acc2

            acc = lax.fori_loop(0, _G, phase_b,
                                [(zerov, zerov)] * _EB)
            for k in range(_EB):
                e = eb * _EB + k
                pk, ck = acc[k]
                plsc.addupdate_scatter(part_v, [lane, ecs[e]], ck)
                plsc.addupdate_scatter(part_v, [lane, ecs[e] + _E], pk)

        pltpu.sync_copy(out_v, combt_hbm.at[:, pl.ds(tok0, _CH)])
        pltpu.sync_copy(idx_v, idx_hbm.at[pl.ds(tok0, _CH)])
        return 0

    lax.fori_loop(0, _NCH, chunk, 0)
    pltpu.sync_copy(part_v, part_hbm.at[wid])


def _aux_tc_body(part_ref, aux_ref):
    x = part_ref[...]
    s = jnp.sum(jnp.sum(x, axis=0), axis=0)
    scale = jnp.float32(_E) / (jnp.float32(_T) * jnp.float32(_T))
    aux_ref[0, 0] = scale * jnp.sum(s[:_E] * s[_E:])


def kernel(router_logits):
    xt = router_logits.T
    combt, idx, part = pl.kernel(
        _router_sc_body,
        out_type=[
            jax.ShapeDtypeStruct((_E, _T), jnp.float32),
            jax.ShapeDtypeStruct((_T,), jnp.int32),
            jax.ShapeDtypeStruct((_NW, _L, 2 * _E), jnp.float32),
        ],
        mesh=plsc.VectorSubcoreMesh(core_axis_name="c", subcore_axis_name="s",
                                    num_cores=_NC, num_subcores=_NS),
        compiler_params=pltpu.CompilerParams(needs_layout_passes=False,
                                             use_tc_tiling_on_sc=True),
        scratch_types=[
            pltpu.VMEM((_E, _CH), jnp.float32),
            pltpu.VMEM((_E, _CH), jnp.float32),
            pltpu.VMEM((_E, _CH), jnp.float32),
            pltpu.VMEM((_CH,), jnp.int32),
            pltpu.VMEM((_G, _L), jnp.float32),
            pltpu.VMEM((_L, 2 * _E), jnp.float32),
        ],
    )(xt)
    aux = pl.pallas_call(
        _aux_tc_body,
        out_shape=jax.ShapeDtypeStruct((1, 1), jnp.float32),
        out_specs=pl.BlockSpec(memory_space=pltpu.SMEM),
    )(part)[0, 0]
    return combt.T, idx, aux


# speedup vs baseline: 1.1064x; 1.1064x over previous
import jax
import jax.numpy as jnp
from jax import lax
from jax.experimental import pallas as pl
from jax.experimental.pallas import tpu as pltpu
from jax.experimental.pallas import tpu_sc as plsc

_T = 32768
_E = 64
_L = 16
_NC = 2
_NS = 16
_NW = _NC * _NS
_TPW = _T // _NW
_CH = 256
_NCH = _TPW // _CH
_G = _CH // _L
_EB = 8
_NGA = 2


def _router_sc_body(xt_hbm, combt_hbm, idx_hbm, part_hbm,
                    in_v, ex_v, out_v, idx_v, si_v, part_v,
                    in_sem, out_sem):
    wid = lax.axis_index("s") * _NC + lax.axis_index("c")
    base = wid * _TPW

    lane = lax.iota(jnp.int32, _L)
    big = jnp.full((_L,), jnp.int32(_E), jnp.int32)
    onev = jnp.full((_L,), jnp.float32(1.0), jnp.float32)
    zerov = jnp.zeros((_L,), jnp.float32)
    ecs = [jnp.full((_L,), jnp.int32(e), jnp.int32) for e in range(_E)]

    for r in range(_L):
        for k in range(2 * _E // _L):
            part_v[r, pl.ds(k * _L, _L)] = zerov

    in_cp = [pltpu.async_copy(xt_hbm.at[:, pl.ds(base, _CH)],
                              in_v.at[0], in_sem)]
    out_cp = []

    for c in range(_NCH):
        tok0 = base + c * _CH
        b = c % 2
        in_cp[c].wait()
        if c + 1 < _NCH:
            in_cp.append(pltpu.async_copy(
                xt_hbm.at[:, pl.ds(tok0 + _CH, _CH)],
                in_v.at[1 - b], in_sem))
        if c >= 2:
            for cp in out_cp[2 * (c - 2):2 * (c - 1)]:
                cp.wait()

        def phase_a(gq, _):
            for q in range(_NGA):
                g = gq * _NGA + q
                t0 = g * _L
                part = []
                for h in range(4):
                    mh = in_v[b, 16 * h, pl.ds(t0, _L)]
                    for e in range(16 * h + 1, 16 * h + 16):
                        mh = jnp.maximum(mh, in_v[b, e, pl.ds(t0, _L)])
                    part.append(mh)
                m = jnp.maximum(jnp.maximum(part[0], part[1]),
                                jnp.maximum(part[2], part[3]))
                ss, bb = [], []
                for h in range(4):
                    sh = zerov
                    bh = big
                    for e in range(16 * h, 16 * h + 16):
                        v = in_v[b, e, pl.ds(t0, _L)]
                        ex = jnp.exp(v - m)
                        ex_v[e, pl.ds(t0, _L)] = ex
                        sh = sh + ex
                        bh = jnp.minimum(bh, jnp.where(v == m, ecs[e], big))
                    ss.append(sh)
                    bb.append(bh)
                s = (ss[0] + ss[1]) + (ss[2] + ss[3])
                bi = jnp.minimum(jnp.minimum(bb[0], bb[1]),
                                 jnp.minimum(bb[2], bb[3]))
                si_v[g, :] = onev / s
                idx_v[b, pl.ds(t0, _L)] = bi
            return 0

        lax.fori_loop(0, _G // _NGA, phase_a, 0)

        for eb in range(_E // _EB):
            def phase_b(g, acc):
                t0 = g * _L
                bi = idx_v[b, pl.ds(t0, _L)]
                sinv = si_v[g, :]
                acc2 = []
                for k in range(_EB):
                    e = eb * _EB + k
                    pk, ck = acc[k]
                    ex = ex_v[e, pl.ds(t0, _L)]
                    hit = bi == ecs[e]
                    out_v[b, e, pl.ds(t0, _L)] = jnp.where(hit, sinv, zerov)
                    acc2.append((pk + ex * sinv,
                                 ck + jnp.where(hit, onev, zerov)))
                return acc2

            acc = lax.fori_loop(0, _G, phase_b,
                                [(zerov, zerov)] * _EB)
            for k in range(_EB):
                e = eb * _EB + k
                pk, ck = acc[k]
                plsc.addupdate_scatter(part_v, [lane, ecs[e]], ck)
                plsc.addupdate_scatter(part_v, [lane, ecs[e] + _E], pk)

        out_cp.append(pltpu.async_copy(
            out_v.at[b], combt_hbm.at[:, pl.ds(tok0, _CH)], out_sem))
        out_cp.append(pltpu.async_copy(
            idx_v.at[b], idx_hbm.at[pl.ds(tok0, _CH)], out_sem))

    for cp in out_cp[2 * (_NCH - 2):]:
        cp.wait()
    pltpu.sync_copy(part_v, part_hbm.at[wid])


def _aux_tc_body(part_ref, aux_ref):
    x = part_ref[...]
    s = jnp.sum(jnp.sum(x, axis=0), axis=0)
    scale = jnp.float32(_E) / (jnp.float32(_T) * jnp.float32(_T))
    aux_ref[0, 0] = scale * jnp.sum(s[:_E] * s[_E:])


def kernel(router_logits):
    xt = router_logits.T
    combt, idx, part = pl.kernel(
        _router_sc_body,
        out_type=[
            jax.ShapeDtypeStruct((_E, _T), jnp.float32),
            jax.ShapeDtypeStruct((_T,), jnp.int32),
            jax.ShapeDtypeStruct((_NW, _L, 2 * _E), jnp.float32),
        ],
        mesh=plsc.VectorSubcoreMesh(core_axis_name="c", subcore_axis_name="s",
                                    num_cores=_NC, num_subcores=_NS),
        compiler_params=pltpu.CompilerParams(needs_layout_passes=False,
                                             use_tc_tiling_on_sc=True),
        scratch_types=[
            pltpu.VMEM((2, _E, _CH), jnp.float32),
            pltpu.VMEM((_E, _CH), jnp.float32),
            pltpu.VMEM((2, _E, _CH), jnp.float32),
            pltpu.VMEM((2, _CH), jnp.int32),
            pltpu.VMEM((_G, _L), jnp.float32),
            pltpu.VMEM((_L, 2 * _E), jnp.float32),
            pltpu.SemaphoreType.DMA,
            pltpu.SemaphoreType.DMA,
        ],
    )(xt)
    aux = pl.pallas_call(
        _aux_tc_body,
        out_shape=jax.ShapeDtypeStruct((1, 1), jnp.float32),
        out_specs=pl.BlockSpec(memory_space=pltpu.SMEM),
    )(part)[0, 0]
    return combt.T, idx, aux


# speedup vs baseline: 1.4797x; 1.3374x over previous
import jax
import jax.numpy as jnp
from jax import lax
from jax.experimental import pallas as pl
from jax.experimental.pallas import tpu as pltpu
from jax.experimental.pallas import tpu_sc as plsc

_T = 32768
_E = 64
_L = 16
_NC = 2
_NS = 16
_NW = _NC * _NS
_TPW = _T // _NW
_CH = 512
_NCH = _TPW // _CH
_G = _CH // _L
_EB = 8
_NGA = 2


def _router_sc_body(xt_hbm, combt_hbm, idx_hbm, part_hbm,
                    in_v, ex_v, out_v, idx_v, si_v, part_v):
    wid = lax.axis_index("s") * _NC + lax.axis_index("c")
    base = wid * _TPW

    lane = lax.iota(jnp.int32, _L)
    big = jnp.full((_L,), jnp.int32(_E), jnp.int32)
    onev = jnp.full((_L,), jnp.float32(1.0), jnp.float32)
    zerov = jnp.zeros((_L,), jnp.float32)
    ecs = [jnp.full((_L,), jnp.int32(e), jnp.int32) for e in range(_E)]

    for r in range(_L):
        for k in range(2 * _E // _L):
            part_v[r, pl.ds(k * _L, _L)] = zerov

    def chunk(c, _):
        tok0 = base + c * _CH
        pltpu.sync_copy(xt_hbm.at[:, pl.ds(tok0, _CH)], in_v)

        def phase_a(gq, _):
            for q in range(_NGA):
                g = gq * _NGA + q
                t0 = g * _L
                part = []
                for h in range(4):
                    mh = in_v[16 * h, pl.ds(t0, _L)]
                    for e in range(16 * h + 1, 16 * h + 16):
                        mh = jnp.maximum(mh, in_v[e, pl.ds(t0, _L)])
                    part.append(mh)
                m = jnp.maximum(jnp.maximum(part[0], part[1]),
                                jnp.maximum(part[2], part[3]))
                ss, bb = [], []
                for h in range(4):
                    sh = zerov
                    bh = big
                    for e in range(16 * h, 16 * h + 16):
                        v = in_v[e, pl.ds(t0, _L)]
                        ex = jnp.exp(v - m)
                        ex_v[e, pl.ds(t0, _L)] = ex
                        sh = sh + ex
                        bh = jnp.minimum(bh, jnp.where(v == m, ecs[e], big))
                    ss.append(sh)
                    bb.append(bh)
                s = (ss[0] + ss[1]) + (ss[2] + ss[3])
                bi = jnp.minimum(jnp.minimum(bb[0], bb[1]),
                                 jnp.minimum(bb[2], bb[3]))
                si_v[g, :] = onev / s
                idx_v[pl.ds(t0, _L)] = bi
            return 0

        lax.fori_loop(0, _G // _NGA, phase_a, 0)

        for eb in range(_E // _EB):
            def phase_b(g, acc):
                t0 = g * _L
                bi = idx_v[pl.ds(t0, _L)]
                sinv = si_v[g, :]
                acc2 = []
                for k in range(_EB):
                    e = eb * _EB + k
                    pk, ck = acc[k]
                    ex = ex_v[e, pl.ds(t0, _L)]
                    hit = bi == ecs[e]
                    out_v[e, pl.ds(t0, _L)] = jnp.where(hit, sinv, zerov)
                    acc2.append((pk + ex * sinv,
                                 ck + jnp.where(hit, onev, zerov)))
                return acc2

            acc = lax.fori_loop(0, _G, phase_b,
                                [(zerov, zerov)] * _EB)
            for k in range(_EB):
                e = eb * _EB + k
                pk, ck = acc[k]
                plsc.addupdate_scatter(part_v, [lane, ecs[e]], ck)
                plsc.addupdate_scatter(part_v, [lane, ecs[e] + _E], pk)

        pltpu.sync_copy(out_v, combt_hbm.at[:, pl.ds(tok0, _CH)])
        pltpu.sync_copy(idx_v, idx_hbm.at[pl.ds(tok0, _CH)])
        return 0

    lax.fori_loop(0, _NCH, chunk, 0)
    pltpu.sync_copy(part_v, part_hbm.at[wid])


def _aux_tc_body(part_ref, aux_ref):
    x = part_ref[...]
    s = jnp.sum(jnp.sum(x, axis=0), axis=0)
    scale = jnp.float32(_E) / (jnp.float32(_T) * jnp.float32(_T))
    aux_ref[0, 0] = scale * jnp.sum(s[:_E] * s[_E:])


def kernel(router_logits):
    xt = router_logits.T
    combt, idx, part = pl.kernel(
        _router_sc_body,
        out_type=[
            jax.ShapeDtypeStruct((_E, _T), jnp.float32),
            jax.ShapeDtypeStruct((_T,), jnp.int32),
            jax.ShapeDtypeStruct((_NW, _L, 2 * _E), jnp.float32),
        ],
        mesh=plsc.VectorSubcoreMesh(core_axis_name="c", subcore_axis_name="s",
                                    num_cores=_NC, num_subcores=_NS),
        compiler_params=pltpu.CompilerParams(needs_layout_passes=False,
                                             use_tc_tiling_on_sc=True),
        scratch_types=[
            pltpu.VMEM((_E, _CH), jnp.float32),
            pltpu.VMEM((_E, _CH), jnp.float32),
            pltpu.VMEM((_E, _CH), jnp.float32),
            pltpu.VMEM((_CH,), jnp.int32),
            pltpu.VMEM((_G, _L), jnp.float32),
            pltpu.VMEM((_L, 2 * _E), jnp.float32),
        ],
    )(xt)
    aux = pl.pallas_call(
        _aux_tc_body,
        out_shape=jax.ShapeDtypeStruct((1, 1), jnp.float32),
        out_specs=pl.BlockSpec(memory_space=pltpu.SMEM),
    )(part)[0, 0]
    return combt.T, idx, aux
